# baseline (speedup 1.0000x reference)
import functools

import jax
import jax.numpy as jnp
from jax import lax
from jax.experimental import pallas as pl
from jax.experimental.pallas import tpu as pltpu
from jax.experimental.pallas import tpu_sc as plsc

BATCH = 16384
EMBED_DIM = 64
LANES = 16
NUM_CORES = 2
NUM_SUBCORES = 16
NUM_WORKERS = NUM_CORES * NUM_SUBCORES
ROWS_PER_WORKER = BATCH // NUM_WORKERS
GATHER_CHUNK = 128
NUM_GCHUNKS = ROWS_PER_WORKER // GATHER_CHUNK
NUM_DCHUNKS = EMBED_DIM // LANES


def _sc_pair_partials(pos_app, pos_entity, neg_app, neg_entity,
                      app_emb, entity_emb):
    mesh = plsc.VectorSubcoreMesh(core_axis_name="c", subcore_axis_name="s")

    @functools.partial(
        pl.kernel,
        out_type=[jax.ShapeDtypeStruct((BATCH, LANES), jnp.float32),
                  jax.ShapeDtypeStruct((BATCH, LANES), jnp.float32)],
        mesh=mesh,
        compiler_params=pltpu.CompilerParams(use_tc_tiling_on_sc=False),
        scratch_types=[
            pltpu.VMEM((ROWS_PER_WORKER,), jnp.int32),
            pltpu.VMEM((ROWS_PER_WORKER,), jnp.int32),
            pltpu.VMEM((ROWS_PER_WORKER, EMBED_DIM), jnp.float32),
            pltpu.VMEM((ROWS_PER_WORKER, EMBED_DIM), jnp.float32),
            pltpu.VMEM((ROWS_PER_WORKER, LANES), jnp.float32),
            pltpu.SemaphoreType.DMA,
        ],
    )
    def sc_kernel(pa_hbm, pe_hbm, na_hbm, ne_hbm, ta_hbm, te_hbm,
                  out_pos, out_neg, idx_a, idx_b, rows_a, rows_b, part, sem):
        wid = lax.axis_index("s") * NUM_CORES + lax.axis_index("c")
        base = wid * ROWS_PER_WORKER

        def do_pair(ia_hbm, ib_hbm, out_hbm):
            pltpu.sync_copy(ia_hbm.at[pl.ds(base, ROWS_PER_WORKER)], idx_a)
            pltpu.sync_copy(ib_hbm.at[pl.ds(base, ROWS_PER_WORKER)], idx_b)
            copies = []
            for c in range(NUM_GCHUNKS):
                rsl = pl.ds(c * GATHER_CHUNK, GATHER_CHUNK)
                copies.append(pltpu.async_copy(
                    ta_hbm.at[idx_a.at[rsl]], rows_a.at[rsl, :], sem))
                copies.append(pltpu.async_copy(
                    te_hbm.at[idx_b.at[rsl]], rows_b.at[rsl, :], sem))
            for cp in copies:
                cp.wait()

            @pl.loop(0, ROWS_PER_WORKER)
            def _(r):
                acc = rows_a[r, pl.ds(0, LANES)] * rows_b[r, pl.ds(0, LANES)]
                for d in range(1, NUM_DCHUNKS):
                    sl = pl.ds(d * LANES, LANES)
                    acc = acc + rows_a[r, sl] * rows_b[r, sl]
                part[r, :] = acc

            pltpu.sync_copy(part, out_hbm.at[pl.ds(base, ROWS_PER_WORKER)])

        do_pair(pa_hbm, pe_hbm, out_pos)
        do_pair(na_hbm, ne_hbm, out_neg)

    return sc_kernel(pos_app, pos_entity, neg_app, neg_entity,
                     app_emb, entity_emb)


def _tc_loss(pos_part, neg_part):
    def body(p_ref, n_ref, o_ref):
        ps = jnp.sum(p_ref[...], axis=1, keepdims=True)
        ns = jnp.sum(n_ref[...], axis=1, keepdims=True)
        pls = jnp.minimum(ps, 0.0) - jnp.log1p(jnp.exp(-jnp.abs(ps)))
        nls = jnp.minimum(-ns, 0.0) - jnp.log1p(jnp.exp(-jnp.abs(ns)))
        o_ref[0, 0] = -(jnp.sum(pls) + jnp.sum(nls))

    out = pl.pallas_call(
        body,
        out_shape=jax.ShapeDtypeStruct((1, 1), jnp.float32),
        out_specs=pl.BlockSpec(memory_space=pltpu.SMEM),
    )(pos_part, neg_part)
    return out[0, 0]


def kernel(pos_app, pos_entity, neg_app, neg_entity, app_emb, entity_emb):
    pos_part, neg_part = _sc_pair_partials(
        pos_app.astype(jnp.int32), pos_entity.astype(jnp.int32),
        neg_app.astype(jnp.int32), neg_entity.astype(jnp.int32),
        app_emb, entity_emb)
    return _tc_loss(pos_part, neg_part)


# speedup vs baseline: 1.5939x; 1.5939x over previous
import functools

import jax
import jax.numpy as jnp
from jax import lax
from jax.experimental import pallas as pl
from jax.experimental.pallas import tpu as pltpu
from jax.experimental.pallas import tpu_sc as plsc

APP_ROWS = 1000000
ENT_ROWS = 1000000
BATCH = 16384
EMBED_DIM = 64
LANES = 16
NUM_CORES = 2
NUM_SUBCORES = 16
NUM_WORKERS = NUM_CORES * NUM_SUBCORES
ROWS_PER_WORKER = BATCH // NUM_WORKERS
CHUNK = 256
NUM_CHUNKS = ROWS_PER_WORKER // CHUNK
NUM_DCHUNKS = EMBED_DIM // LANES
PART_PER_WORKER = ROWS_PER_WORKER * LANES
PART_TOTAL = BATCH * LANES


def _sc_pair_partials(pos_app, pos_entity, neg_app, neg_entity,
                      app_emb, entity_emb):
    mesh = plsc.VectorSubcoreMesh(core_axis_name="c", subcore_axis_name="s")

    @functools.partial(
        pl.kernel,
        out_type=[jax.ShapeDtypeStruct((PART_TOTAL,), jnp.float32),
                  jax.ShapeDtypeStruct((PART_TOTAL,), jnp.float32)],
        mesh=mesh,
        scratch_types=[
            pltpu.VMEM((ROWS_PER_WORKER,), jnp.int32),
            pltpu.VMEM((ROWS_PER_WORKER,), jnp.int32),
            pltpu.VMEM((CHUNK // 8, 8, EMBED_DIM), jnp.float32),
            pltpu.VMEM((CHUNK // 8, 8, EMBED_DIM), jnp.float32),
            pltpu.VMEM((PART_PER_WORKER,), jnp.float32),
            pltpu.SemaphoreType.DMA,
        ],
    )
    def sc_kernel(pa_hbm, pe_hbm, na_hbm, ne_hbm, ta_hbm, te_hbm,
                  out_pos, out_neg, idx_a, idx_b, rows_a, rows_b,
                  part, sem):
        wid = lax.axis_index("s") * NUM_CORES + lax.axis_index("c")
        base = wid * ROWS_PER_WORKER
        tav = ta_hbm.reshape(APP_ROWS // 8, 8, EMBED_DIM)
        tev = te_hbm.reshape(ENT_ROWS // 8, 8, EMBED_DIM)

        def do_pair(ia_hbm, ib_hbm, out_hbm):
            pltpu.sync_copy(ia_hbm.at[pl.ds(base, ROWS_PER_WORKER)], idx_a)
            pltpu.sync_copy(ib_hbm.at[pl.ds(base, ROWS_PER_WORKER)], idx_b)

            for c in range(NUM_CHUNKS):
                cbase = c * CHUNK

                @pl.loop(0, CHUNK // LANES)
                def _(g):
                    iva = idx_a[pl.ds(cbase + g * LANES, LANES)]
                    ivb = idx_b[pl.ds(cbase + g * LANES, LANES)]
                    for k in range(LANES):
                        q = g * 2 + k // 8
                        ia = iva[k]
                        ib = ivb[k]
                        pltpu.async_copy(tav.at[ia >> 3, ia & 7],
                                         rows_a.at[q, k % 8], sem)
                        pltpu.async_copy(tev.at[ib >> 3, ib & 7],
                                         rows_b.at[q, k % 8], sem)

                @pl.loop(0, CHUNK)
                def _(j):
                    pltpu.make_async_copy(tav.at[0, 0], rows_a.at[0, 0],
                                          sem).wait()
                    pltpu.make_async_copy(tev.at[0, 0], rows_b.at[0, 0],
                                          sem).wait()

                @pl.loop(0, CHUNK // 8)
                def _(q):
                    for s in range(8):
                        acc = (rows_a[q, s, pl.ds(0, LANES)] *
                               rows_b[q, s, pl.ds(0, LANES)])
                        for d in range(1, NUM_DCHUNKS):
                            sl = pl.ds(d * LANES, LANES)
                            acc = acc + rows_a[q, s, sl] * rows_b[q, s, sl]
                        part[pl.ds((cbase + q * 8 + s) * LANES, LANES)] = acc

            pltpu.sync_copy(part, out_hbm.at[pl.ds(wid * PART_PER_WORKER,
                                                   PART_PER_WORKER)])

        do_pair(pa_hbm, pe_hbm, out_pos)
        do_pair(na_hbm, ne_hbm, out_neg)

    return sc_kernel(pos_app, pos_entity, neg_app, neg_entity,
                     app_emb, entity_emb)


def _tc_loss(pos_part, neg_part):
    def body(p_ref, n_ref, o_ref):
        lane = lax.broadcasted_iota(jnp.int32, (128, 8), 0)
        grp = lax.broadcasted_iota(jnp.int32, (128, 8), 1)
        sel = (lane // LANES == grp).astype(jnp.float32)
        dn = (((1,), (0,)), ((), ()))
        ps = lax.dot_general(p_ref[...], sel, dn,
                             preferred_element_type=jnp.float32)
        ns = lax.dot_general(n_ref[...], sel, dn,
                             preferred_element_type=jnp.float32)
        pls = jnp.minimum(ps, 0.0) - jnp.log1p(jnp.exp(-jnp.abs(ps)))
        nls = jnp.minimum(-ns, 0.0) - jnp.log1p(jnp.exp(-jnp.abs(ns)))
        o_ref[0, 0] = -(jnp.sum(pls) + jnp.sum(nls))

    out = pl.pallas_call(
        body,
        out_shape=jax.ShapeDtypeStruct((1, 1), jnp.float32),
        out_specs=pl.BlockSpec(memory_space=pltpu.SMEM),
    )(pos_part.reshape(PART_TOTAL // 128, 128),
      neg_part.reshape(PART_TOTAL // 128, 128))
    return out[0, 0]


def kernel(pos_app, pos_entity, neg_app, neg_entity, app_emb, entity_emb):
    pos_part, neg_part = _sc_pair_partials(
        pos_app.astype(jnp.int32), pos_entity.astype(jnp.int32),
        neg_app.astype(jnp.int32), neg_entity.astype(jnp.int32),
        app_emb, entity_emb)
    return _tc_loss(pos_part, neg_part)
